# baseline (speedup 1.0000x reference)
import functools

import jax
import jax.numpy as jnp
from jax import lax
from jax.experimental import pallas as pl
from jax.experimental.pallas import tpu as pltpu
from jax.experimental.pallas import tpu_sc as plsc

N_NODES = 10000
N_EDGES = 320000
D = 128


_ROWS_BLK = 400


def _mlp_body(x_ref, w1t_ref, b1_ref, w2t_ref, b2_ref, o_ref):
    h = jnp.dot(x_ref[...], w1t_ref[...], preferred_element_type=jnp.float32)
    h = h + b1_ref[...]
    o = jnp.dot(h, w2t_ref[...], preferred_element_type=jnp.float32)
    o = o + b2_ref[...]
    lo = jax.lax.bitcast_convert_type(o[:, :64].astype(jnp.bfloat16),
                                      jnp.uint16)
    hi = jax.lax.bitcast_convert_type(o[:, 64:].astype(jnp.bfloat16),
                                      jnp.uint16)
    o_ref[...] = lo.astype(jnp.int32) | (hi.astype(jnp.int32) << 16)


def _mlp(x, w1t, b1, w2t, b2):
    grid = (N_NODES // _ROWS_BLK,)
    return pl.pallas_call(
        _mlp_body,
        grid=grid,
        in_specs=[
            pl.BlockSpec((_ROWS_BLK, D), lambda i: (i, 0)),
            pl.BlockSpec((D, D), lambda i: (0, 0)),
            pl.BlockSpec((1, D), lambda i: (0, 0)),
            pl.BlockSpec((D, D), lambda i: (0, 0)),
            pl.BlockSpec((1, D), lambda i: (0, 0)),
        ],
        out_specs=pl.BlockSpec((_ROWS_BLK, D // 2), lambda i: (i, 0)),
        out_shape=jax.ShapeDtypeStruct((N_NODES, D // 2), jnp.int32),
    )(x, w1t, b1, w2t, b2)


_NC = 2
_NS = 16
_NW = _NC * _NS
_EPW = N_EDGES // _NW
_CH = 80
_NCHUNK = _EPW // _CH
_GRP = _CH // 16


_NBUF = 4


_PSTRIDE = 17


_RPT = N_NODES // _NS


def _edge_dot_body(h_hbm, eli_hbm, out_hbm,
                   sidx_v, didx_v, srows, drows, out_v, pscr, hsp,
                   sems, semd):
    sid = lax.axis_index("s")
    wid = sid * _NC + lax.axis_index("c")
    base = wid * _EPW
    pltpu.sync_copy(eli_hbm.at[0, pl.ds(base, _EPW)], sidx_v)
    pltpu.sync_copy(eli_hbm.at[1, pl.ds(base, _EPW)], didx_v)
    pltpu.sync_copy(h_hbm.at[pl.ds(sid * _RPT, _RPT)],
                    hsp.at[pl.ds(sid * _RPT, _RPT)])
    plsc.subcore_barrier()

    lane = lax.iota(jnp.int32, 16)

    def start(j, b):
        cbase = j * _CH
        pltpu.async_copy(hsp.at[sidx_v.at[pl.ds(cbase, _CH)]],
                         srows.at[b], sems[b])
        pltpu.async_copy(hsp.at[didx_v.at[pl.ds(cbase, _CH)]],
                         drows.at[b], semd[b])

    def wait(b):
        pltpu.make_async_copy(hsp.at[sidx_v.at[pl.ds(0, _CH)]],
                              srows.at[b], sems[b]).wait()
        pltpu.make_async_copy(hsp.at[didx_v.at[pl.ds(0, _CH)]],
                              drows.at[b], semd[b]).wait()

    lane17 = lane * _PSTRIDE

    def compute(j, b):
        cbase = j * _CH
        sb = srows.at[b]
        db = drows.at[b]

        def group_body(g, carry2):
            partials = []
            for e in range(16):
                row = g * 16 + e
                acc = None
                for k in range(D // 32):
                    sv = sb[row, pl.ds(k * 16, 16)]
                    dv = db[row, pl.ds(k * 16, 16)]
                    pbf = (plsc.bitcast(sv, jnp.bfloat16) *
                           plsc.bitcast(dv, jnp.bfloat16))
                    pa, pb = plsc.unpack(
                        pbf, format=plsc.PackFormat.INTERLEAVED,
                        preferred_element_type=jnp.float32)
                    p = pa + pb
                    acc = p if acc is None else acc + p
                partials.append(acc)
            for e in range(16):
                plsc.store_scatter(pscr, [lane + e * _PSTRIDE], partials[e])
            tot = None
            for k in range(16):
                ck = plsc.load_gather(pscr, [lane17 + k])
                tot = ck if tot is None else tot + ck
            out_v[pl.ds(cbase + g * 16, 16)] = tot
            return carry2

        lax.fori_loop(0, _GRP, group_body, 0)

    _MAIN = _NCHUNK - (_NCHUNK % _NBUF)

    for b in range(_NBUF):
        start(b, b)

    @pl.loop(0, _MAIN, step=_NBUF)
    def chunk_body(j0):
        for b in range(_NBUF):
            j = j0 + b
            wait(b)
            compute(j, b)
            nxt = j + _NBUF

            @pl.when(nxt < _MAIN)
            def _():
                start(nxt, b)

    for t in range(_NCHUNK % _NBUF):
        start(_MAIN + t, t)
        wait(t)
        compute(_MAIN + t, t)

    pltpu.sync_copy(out_v, out_hbm.at[pl.ds(base, _EPW)])


def _edge_dot(h, eli):
    mesh = plsc.VectorSubcoreMesh(core_axis_name="c", subcore_axis_name="s")
    f = pl.kernel(
        _edge_dot_body,
        out_type=jax.ShapeDtypeStruct((N_EDGES,), jnp.float32),
        mesh=mesh,
        scratch_types=[
            pltpu.VMEM((_EPW,), jnp.int32),
            pltpu.VMEM((_EPW,), jnp.int32),
            pltpu.VMEM((_NBUF, _CH, D // 2), jnp.int32),
            pltpu.VMEM((_NBUF, _CH, D // 2), jnp.int32),
            pltpu.VMEM((_EPW,), jnp.float32),
            pltpu.VMEM((16 * _PSTRIDE,), jnp.float32),
            pltpu.VMEM_SHARED((N_NODES, D // 2), jnp.int32),
            [pltpu.SemaphoreType.DMA] * _NBUF,
            [pltpu.SemaphoreType.DMA] * _NBUF,
        ],
        compiler_params=pltpu.CompilerParams(needs_layout_passes=False,
                                             use_tc_tiling_on_sc=False),
    )
    return f(h, eli)


def kernel(x, edge_label_index, W1, b1, W2, b2):
    h_pack = _mlp(x, W1.T, b1.reshape(1, D), W2.T, b2.reshape(1, D))
    return _edge_dot(h_pack, edge_label_index.astype(jnp.int32))


# speedup vs baseline: 1.1351x; 1.1351x over previous
import functools

import jax
import jax.numpy as jnp
from jax import lax
from jax.experimental import pallas as pl
from jax.experimental.pallas import tpu as pltpu
from jax.experimental.pallas import tpu_sc as plsc

N_NODES = 10000
N_EDGES = 320000
D = 128


def _mlp_body(x_ref, w1t_ref, b1_ref, w2t_ref, b2_ref, o_ref):
    w = jnp.dot(w1t_ref[...], w2t_ref[...], preferred_element_type=jnp.float32)
    bb = jnp.dot(b1_ref[...], w2t_ref[...], preferred_element_type=jnp.float32)
    o = jnp.dot(x_ref[...], w, preferred_element_type=jnp.float32)
    o = o + (bb + b2_ref[...])
    lo = jax.lax.bitcast_convert_type(o[:, :64].astype(jnp.bfloat16),
                                      jnp.uint16)
    hi = jax.lax.bitcast_convert_type(o[:, 64:].astype(jnp.bfloat16),
                                      jnp.uint16)
    o_ref[...] = lo.astype(jnp.int32) | (hi.astype(jnp.int32) << 16)


def _mlp(x, w1t, b1, w2t, b2):
    return pl.pallas_call(
        _mlp_body,
        out_shape=jax.ShapeDtypeStruct((N_NODES, D // 2), jnp.int32),
    )(x, w1t, b1, w2t, b2)


_NC = 2
_NS = 16
_NW = _NC * _NS
_EPW = N_EDGES // _NW
_CH = 80
_NCHUNK = _EPW // _CH
_GRP = _CH // 16


_NBUF = 4


_PSTRIDE = 17


def _edge_dot_body(h_hbm, eli_hbm, out_hbm,
                   sidx_v, didx_v, srows, drows, out_v, pscr, sems, semd):
    wid = lax.axis_index("s") * _NC + lax.axis_index("c")
    base = wid * _EPW
    pltpu.sync_copy(eli_hbm.at[0, pl.ds(base, _EPW)], sidx_v)
    pltpu.sync_copy(eli_hbm.at[1, pl.ds(base, _EPW)], didx_v)

    lane = lax.iota(jnp.int32, 16)

    def start(j, b):
        cbase = j * _CH
        pltpu.async_copy(h_hbm.at[sidx_v.at[pl.ds(cbase, _CH)]],
                         srows.at[b], sems[b])
        pltpu.async_copy(h_hbm.at[didx_v.at[pl.ds(cbase, _CH)]],
                         drows.at[b], semd[b])

    def wait(b):
        pltpu.make_async_copy(h_hbm.at[sidx_v.at[pl.ds(0, _CH)]],
                              srows.at[b], sems[b]).wait()
        pltpu.make_async_copy(h_hbm.at[didx_v.at[pl.ds(0, _CH)]],
                              drows.at[b], semd[b]).wait()

    lane17 = lane * _PSTRIDE

    def compute(j, b):
        cbase = j * _CH
        sb = srows.at[b]
        db = drows.at[b]

        def group_body(g, carry2):
            partials = []
            for e in range(16):
                row = g * 16 + e
                acc = None
                for k in range(D // 32):
                    sv = sb[row, pl.ds(k * 16, 16)]
                    dv = db[row, pl.ds(k * 16, 16)]
                    pbf = (plsc.bitcast(sv, jnp.bfloat16) *
                           plsc.bitcast(dv, jnp.bfloat16))
                    pa, pb = plsc.unpack(
                        pbf, format=plsc.PackFormat.INTERLEAVED,
                        preferred_element_type=jnp.float32)
                    p = pa + pb
                    acc = p if acc is None else acc + p
                partials.append(acc)
            for e in range(16):
                plsc.store_scatter(pscr, [lane + e * _PSTRIDE], partials[e])
            tot = None
            for k in range(16):
                ck = plsc.load_gather(pscr, [lane17 + k])
                tot = ck if tot is None else tot + ck
            out_v[pl.ds(cbase + g * 16, 16)] = tot
            return carry2

        lax.fori_loop(0, _GRP, group_body, 0)

    _MAIN = _NCHUNK - (_NCHUNK % _NBUF)

    for b in range(_NBUF):
        start(b, b)

    @pl.loop(0, _MAIN, step=_NBUF)
    def chunk_body(j0):
        for b in range(_NBUF):
            j = j0 + b
            wait(b)
            compute(j, b)
            nxt = j + _NBUF

            @pl.when(nxt < _MAIN)
            def _():
                start(nxt, b)

    for t in range(_NCHUNK % _NBUF):
        start(_MAIN + t, t)
        wait(t)
        compute(_MAIN + t, t)

    pltpu.sync_copy(out_v, out_hbm.at[pl.ds(base, _EPW)])


def _edge_dot(h, eli):
    mesh = plsc.VectorSubcoreMesh(core_axis_name="c", subcore_axis_name="s")
    f = pl.kernel(
        _edge_dot_body,
        out_type=jax.ShapeDtypeStruct((N_EDGES,), jnp.float32),
        mesh=mesh,
        scratch_types=[
            pltpu.VMEM((_EPW,), jnp.int32),
            pltpu.VMEM((_EPW,), jnp.int32),
            pltpu.VMEM((_NBUF, _CH, D // 2), jnp.int32),
            pltpu.VMEM((_NBUF, _CH, D // 2), jnp.int32),
            pltpu.VMEM((_EPW,), jnp.float32),
            pltpu.VMEM((16 * _PSTRIDE,), jnp.float32),
            [pltpu.SemaphoreType.DMA] * _NBUF,
            [pltpu.SemaphoreType.DMA] * _NBUF,
        ],
        compiler_params=pltpu.CompilerParams(needs_layout_passes=False,
                                             use_tc_tiling_on_sc=False),
    )
    return f(h, eli)


def kernel(x, edge_label_index, W1, b1, W2, b2):
    h_pack = _mlp(x, W1.T, b1.reshape(1, D), W2.T, b2.reshape(1, D))
    return _edge_dot(h_pack, edge_label_index.astype(jnp.int32))
